# baseline (speedup 1.0000x reference)
import functools

import jax
import jax.numpy as jnp
from jax import lax
from jax.experimental import pallas as pl
from jax.experimental.pallas import tpu as pltpu
from jax.experimental.pallas import tpu_sc as plsc

HIDDEN = 64
FIELD_A = 1000
BATCH = 16384

_NC, _NS, _L = 2, 16, 16
_NW = _NC * _NS
_B_PER_W = BATCH // _NW


def _gather_kernel(idx_a_hbm, idx_b_hbm, table_hbm, out_hbm,
                   idx_v, idx_b_v, rows_v, sem):
    wid = lax.axis_index("s") * _NC + lax.axis_index("c")
    base = wid * _B_PER_W

    pltpu.sync_copy(idx_a_hbm.at[pl.ds(base, _B_PER_W)], idx_v)
    pltpu.sync_copy(idx_b_hbm.at[pl.ds(base, _B_PER_W)], idx_b_v)

    for i in range(_B_PER_W // _L):
        s = pl.ds(i * _L, _L)
        idx_v[s] = idx_v[s] + idx_b_v[s] * FIELD_A

    pltpu.async_copy(table_hbm.at[idx_v], rows_v, sem).wait()
    pltpu.sync_copy(rows_v, out_hbm.at[pl.ds(base, _B_PER_W)])


def kernel(idx_a, idx_b, table):
    mesh = plsc.VectorSubcoreMesh(core_axis_name="c", subcore_axis_name="s")
    k = functools.partial(
        pl.kernel,
        mesh=mesh,
        out_type=jax.ShapeDtypeStruct((BATCH, HIDDEN), jnp.float32),
        scratch_types=[
            pltpu.VMEM((_B_PER_W,), jnp.int32),
            pltpu.VMEM((_B_PER_W,), jnp.int32),
            pltpu.VMEM((_B_PER_W, HIDDEN), jnp.float32),
            pltpu.SemaphoreType.DMA,
        ],
        compiler_params=pltpu.CompilerParams(use_tc_tiling_on_sc=False),
    )(_gather_kernel)
    return k(idx_a, idx_b, table)


# speedup vs baseline: 1.0295x; 1.0295x over previous
import functools

import jax
import jax.numpy as jnp
from jax import lax
from jax.experimental import pallas as pl
from jax.experimental.pallas import tpu as pltpu
from jax.experimental.pallas import tpu_sc as plsc

HIDDEN = 64
FIELD_A = 1000
BATCH = 16384
TABLE_ROWS = 1000000

_NC, _NS, _L = 2, 16, 16
_NW = _NC * _NS
_B_PER_W = BATCH // _NW


def _gather_kernel(idx_a_hbm, idx_b_hbm, table_hbm, out_hbm,
                   idx_av, idx_bv, eidx_v, sem):
    wid = lax.axis_index("s") * _NC + lax.axis_index("c")
    base = wid * _B_PER_W

    pltpu.sync_copy(idx_a_hbm.at[pl.ds(base, _B_PER_W)], idx_av)
    pltpu.sync_copy(idx_b_hbm.at[pl.ds(base, _B_PER_W)], idx_bv)

    for i in range(_B_PER_W // _L):
        s = pl.ds(i * _L, _L)
        eidx_v[s] = idx_av[s] + idx_bv[s] * FIELD_A

    lane = lax.iota(jnp.int32, _L)

    def _fire(g, _):
        vs = eidx_v[pl.ds(g * _L, _L)]
        for l in range(_L):
            e = jnp.max(jnp.where(lane == l, vs, -1))
            pltpu.make_async_copy(
                table_hbm.at[pl.ds(e, 1)],
                out_hbm.at[pl.ds(base + g * _L + l, 1)],
                sem,
            ).start()
        return _

    lax.fori_loop(0, _B_PER_W // _L, _fire, 0)

    def _drain(j, _):
        pltpu.make_async_copy(
            table_hbm.at[pl.ds(0, 1)], out_hbm.at[pl.ds(base, 1)], sem
        ).wait()
        return _

    lax.fori_loop(0, _B_PER_W, _drain, 0)


def kernel(idx_a, idx_b, table):
    mesh = plsc.VectorSubcoreMesh(core_axis_name="c", subcore_axis_name="s")
    k = functools.partial(
        pl.kernel,
        mesh=mesh,
        out_type=jax.ShapeDtypeStruct((BATCH, HIDDEN), jnp.float32),
        scratch_types=[
            pltpu.VMEM((_B_PER_W,), jnp.int32),
            pltpu.VMEM((_B_PER_W,), jnp.int32),
            pltpu.VMEM((_B_PER_W,), jnp.int32),
            pltpu.SemaphoreType.DMA,
        ],
        compiler_params=pltpu.CompilerParams(needs_layout_passes=False),
    )(_gather_kernel)
    return k(idx_a, idx_b, table)
